# baseline (speedup 1.0000x reference)
import functools

import jax
import jax.numpy as jnp
from jax import lax
from jax.experimental import pallas as pl
from jax.experimental.pallas import tpu as pltpu
from jax.experimental.pallas import tpu_sc as plsc


def _ln(x, g, b):
    mu = jnp.mean(x, axis=-1, keepdims=True)
    var = jnp.mean((x - mu) ** 2, axis=-1, keepdims=True)
    return (x - mu) / jnp.sqrt(var + 1e-5) * g + b


def _mm(a, b):
    return jax.lax.dot_general(a, b, (((1,), (0,)), ((), ())),
                               preferred_element_type=jnp.float32)


def _top3(sxyz, xyzT):
    d = None
    for c in range(3):
        diff = sxyz[:, c:c + 1] - xyzT[c:c + 1, :]
        sq = diff * diff
        d = sq if d is None else d + sq
    m_count = float(xyzT.shape[1])
    iota = jax.lax.broadcasted_iota(jnp.int32, d.shape, 1).astype(jnp.float32)
    remaining = d
    idxs, ws = [], []
    wsum = None
    for r in range(3):
        mval = jnp.min(remaining, axis=1, keepdims=True)
        eq = remaining == mval
        idx = jnp.min(jnp.where(eq, iota, m_count), axis=1, keepdims=True)
        dist = jnp.sqrt(jnp.maximum(mval, 1e-10))
        w = 1.0 / (dist + 1e-8)
        wsum = w if r == 0 else wsum + w
        idxs.append(idx)
        ws.append(w)
        if r < 2:
            remaining = jnp.where(iota == idx, jnp.inf, remaining)
    return (jnp.concatenate(idxs, axis=1).astype(jnp.int32),
            jnp.concatenate([w / wsum for w in ws], axis=1))


def _knn_mix(sxyz, xyzT, f2):
    idx, w = _top3(sxyz, xyzT)
    m_count = xyzT.shape[1]
    iota = jax.lax.broadcasted_iota(jnp.int32, (sxyz.shape[0], m_count), 1)
    sel_w = jnp.zeros((sxyz.shape[0], m_count), jnp.float32)
    for r in range(3):
        sel_w = sel_w + jnp.where(iota == idx[:, r:r + 1], w[:, r:r + 1], 0.0)
    return _mm(sel_w, f2)


def _fused_b_body(feats, xyz0T, sxyz0, sfeats0, xyz1T, sxyz1, sfeats1,
                  xyz2T, sxyz2, sfeats2,
                  u0g1, u0b1, u0w1, u0bb1, u0g2, u0b2, u0w2, u0bb2,
                  u1g1, u1b1, u1w1, u1bb1, u1g2, u1b2, u1w2, u1bb2,
                  u2g1, u2b1, u2w1, u2bb1, u2g2, u2b2, u2w2, u2bb2,
                  u3g2, u3b2, u3w2, u3bb2,
                  sxyz3, xyz3T, sfeats3, g1, b1, w1, bb1,
                  f2_3_out, a_out, idx_out, w_out):
    @pl.when(pl.program_id(0) == 0)
    def _stages012():
        f = feats[...]
        params = (
            (xyz0T, sxyz0, sfeats0, u0g1, u0b1, u0w1, u0bb1, u0g2, u0b2, u0w2, u0bb2),
            (xyz1T, sxyz1, sfeats1, u1g1, u1b1, u1w1, u1bb1, u1g2, u1b2, u1w2, u1bb2),
            (xyz2T, sxyz2, sfeats2, u2g1, u2b1, u2w1, u2bb1, u2g2, u2b2, u2w2, u2bb2),
        )
        for (xyzT, sxyz, sfeats, sg1, sb1, sw1, sbb1, sg2, sb2, sw2, sbb2) in params:
            a = _mm(_ln(sfeats[...], sg1[...], sb1[...]), sw1[...]) + sbb1[...]
            f2 = _mm(_ln(f, sg2[...], sb2[...]), sw2[...]) + sbb2[...]
            f = a + _knn_mix(sxyz[...], xyzT[...], f2)
        f2_3_out[...] = _mm(_ln(f, u3g2[...], u3b2[...]), u3w2[...]) + u3bb2[...]

    a_out[...] = _mm(_ln(sfeats3[...], g1[...], b1[...]), w1[...]) + bb1[...]
    idx, w = _top3(sxyz3[...], xyz3T[...])
    idx_out[...] = idx
    w_out[...] = w


_P = 40
_NW = 32


def _sc_gather(idx_flat, f2):
    rows_total = idx_flat.shape[0]
    wr = rows_total // _NW
    sub = wr // (3 * _P)

    def body(idx_hbm, f2_hbm, out_hbm, idx_v, rows_v, gsem, ssem):
        wid = lax.axis_index("s") * 2 + lax.axis_index("c")
        base = wid * wr
        pltpu.sync_copy(idx_hbm.at[pl.ds(base, wr)], idx_v)
        gathers = []
        for s in range(sub):
            gathers.append(pltpu.async_copy(
                f2_hbm.at[idx_v.at[pl.ds(s * 3 * _P, 3 * _P)]],
                rows_v.at[pl.ds(s * 3 * _P, 3 * _P)], gsem))
        stores = []
        for s in range(sub):
            gathers[s].wait()
            stores.append(pltpu.async_copy(
                rows_v.at[pl.ds(s * 3 * _P, 3 * _P)],
                out_hbm.at[pl.ds(base + s * 3 * _P, 3 * _P)], ssem))
        for c in stores:
            c.wait()

    return functools.partial(
        pl.kernel,
        mesh=plsc.VectorSubcoreMesh(core_axis_name="c", subcore_axis_name="s"),
        out_type=jax.ShapeDtypeStruct((rows_total, 128), jnp.float32),
        scratch_types=[
            pltpu.VMEM((wr,), jnp.int32),
            pltpu.VMEM((wr, 128), jnp.float32),
            pltpu.SemaphoreType.DMA,
            pltpu.SemaphoreType.DMA,
        ],
    )(body)(idx_flat, f2)


def _combine_body(a_ref, rows_ref, w_ref, cw1, cb1, h_out):
    rows = rows_ref[...]
    w = w_ref[...]
    g = (w[:, 0:1] * rows[:, 0:128]
         + w[:, 1:2] * rows[:, 128:256]
         + w[:, 2:3] * rows[:, 256:384])
    f = a_ref[...] + g
    h_out[...] = _mm(f, cw1[...]) + cb1[...]


def _conf_body(h_ref, bn_g, bn_b, w2T, b2, conf_out):
    h = h_ref[...]
    mu = jnp.mean(h, axis=0, keepdims=True)
    var = jnp.mean((h - mu) ** 2, axis=0, keepdims=True)
    h = (h - mu) / jnp.sqrt(var + 1e-5) * bn_g[...] + bn_b[...]
    h = jnp.maximum(h, 0.0)
    conf_out[...] = jnp.sum(h * w2T[...], axis=1, keepdims=True) + b2[...]


_BN3 = 1000


def kernel(feats, xyz0, sxyz0, sfeats0, xyz1, sxyz1, sfeats1, xyz2, sxyz2, sfeats2, xyz3, sxyz3, sfeats3, u0_ln1_g, u0_ln1_b, u0_w1, u0_b1, u0_ln2_g, u0_ln2_b, u0_w2, u0_b2, u1_ln1_g, u1_ln1_b, u1_w1, u1_b1, u1_ln2_g, u1_ln2_b, u1_w2, u1_b2, u2_ln1_g, u2_ln1_b, u2_w1, u2_b1, u2_ln2_g, u2_ln2_b, u2_w2, u2_b2, u3_ln1_g, u3_ln1_b, u3_w1, u3_b1, u3_ln2_g, u3_ln2_b, u3_w2, u3_b2, conf_w1, conf_b1, conf_bn_g, conf_bn_b, conf_w2, conf_b2):
    r1 = lambda v: v.reshape(1, -1)
    n3 = sxyz3.shape[0]
    grid = (n3 // _BN3,)
    full = lambda s: pl.BlockSpec(s, lambda i: (0,) * len(s))
    blk = lambda s: pl.BlockSpec(s, lambda i: (i,) + (0,) * (len(s) - 1))

    f2_3, a3, idx3, w3 = pl.pallas_call(
        _fused_b_body,
        grid=grid,
        in_specs=[
            full((39, 512)),
            full((3, 39)), full((156, 3)), full((156, 512)),
            full((3, 156)), full((625, 3)), full((625, 512)),
            full((3, 625)), full((2500, 3)), full((2500, 256)),
            full((1, 512)), full((1, 512)), full((512, 512)), full((1, 512)),
            full((1, 512)), full((1, 512)), full((512, 512)), full((1, 512)),
            full((1, 512)), full((1, 512)), full((512, 512)), full((1, 512)),
            full((1, 512)), full((1, 512)), full((512, 512)), full((1, 512)),
            full((1, 256)), full((1, 256)), full((256, 256)), full((1, 256)),
            full((1, 512)), full((1, 512)), full((512, 256)), full((1, 256)),
            full((1, 256)), full((1, 256)), full((256, 128)), full((1, 128)),
            blk((_BN3, 3)),
            full((3, 2500)),
            blk((_BN3, 128)),
            full((1, 128)), full((1, 128)),
            full((128, 128)), full((1, 128)),
        ],
        out_specs=[full((2500, 128)),
                   blk((_BN3, 128)), blk((_BN3, 3)), blk((_BN3, 3))],
        out_shape=[jax.ShapeDtypeStruct((2500, 128), jnp.float32),
                   jax.ShapeDtypeStruct((n3, 128), jnp.float32),
                   jax.ShapeDtypeStruct((n3, 3), jnp.int32),
                   jax.ShapeDtypeStruct((n3, 3), jnp.float32)],
    )(feats, xyz0.T, sxyz0, sfeats0, xyz1.T, sxyz1, sfeats1, xyz2.T, sxyz2,
      sfeats2,
      r1(u0_ln1_g), r1(u0_ln1_b), u0_w1, r1(u0_b1), r1(u0_ln2_g), r1(u0_ln2_b), u0_w2, r1(u0_b2),
      r1(u1_ln1_g), r1(u1_ln1_b), u1_w1, r1(u1_b1), r1(u1_ln2_g), r1(u1_ln2_b), u1_w2, r1(u1_b2),
      r1(u2_ln1_g), r1(u2_ln1_b), u2_w1, r1(u2_b1), r1(u2_ln2_g), r1(u2_ln2_b), u2_w2, r1(u2_b2),
      r1(u3_ln2_g), r1(u3_ln2_b), u3_w2, r1(u3_b2),
      sxyz3, xyz3.T, sfeats3, r1(u3_ln1_g), r1(u3_ln1_b), u3_w1, r1(u3_b1))

    rows_pad = -(3 * n3) % (_NW * 3 * _P)
    npad = (3 * n3 + rows_pad) // 3
    idx_flat = jnp.pad(idx3.reshape(-1), (0, rows_pad))
    rows = _sc_gather(idx_flat, f2_3).reshape(npad, 384)

    h = pl.pallas_call(
        _combine_body,
        grid=grid,
        in_specs=[
            blk((_BN3, 128)),
            blk((_BN3, 384)),
            blk((_BN3, 3)),
            full((128, 128)), full((1, 128)),
        ],
        out_specs=blk((_BN3, 128)),
        out_shape=jax.ShapeDtypeStruct((n3, 128), jnp.float32),
    )(a3, rows, w3, conf_w1, r1(conf_b1))

    conf = pl.pallas_call(
        _conf_body,
        out_shape=jax.ShapeDtypeStruct((n3, 1), jnp.float32),
    )(h, r1(conf_bn_g), r1(conf_bn_b), conf_w2.T, r1(conf_b2))
    return conf


# speedup vs baseline: 1.0385x; 1.0385x over previous
import functools

import jax
import jax.numpy as jnp
from jax import lax
from jax.experimental import pallas as pl
from jax.experimental.pallas import tpu as pltpu
from jax.experimental.pallas import tpu_sc as plsc


def _ln(x, g, b):
    mu = jnp.mean(x, axis=-1, keepdims=True)
    var = jnp.mean((x - mu) ** 2, axis=-1, keepdims=True)
    return (x - mu) / jnp.sqrt(var + 1e-5) * g + b


def _mm(a, b):
    return jax.lax.dot_general(a, b, (((1,), (0,)), ((), ())),
                               preferred_element_type=jnp.float32)


def _top3(sxyz, xyzT):
    d = None
    for c in range(3):
        diff = sxyz[:, c:c + 1] - xyzT[c:c + 1, :]
        sq = diff * diff
        d = sq if d is None else d + sq
    m_count = float(xyzT.shape[1])
    iota = jax.lax.broadcasted_iota(jnp.int32, d.shape, 1).astype(jnp.float32)
    remaining = d
    idxs, ws = [], []
    wsum = None
    for r in range(3):
        mval = jnp.min(remaining, axis=1, keepdims=True)
        eq = remaining == mval
        idx = jnp.min(jnp.where(eq, iota, m_count), axis=1, keepdims=True)
        dist = jnp.sqrt(jnp.maximum(mval, 1e-10))
        w = 1.0 / (dist + 1e-8)
        wsum = w if r == 0 else wsum + w
        idxs.append(idx)
        ws.append(w)
        if r < 2:
            remaining = jnp.where(iota == idx, jnp.inf, remaining)
    return (jnp.concatenate(idxs, axis=1).astype(jnp.int32),
            jnp.concatenate([w / wsum for w in ws], axis=1))


def _knn_mix(sxyz, xyzT, f2):
    idx, w = _top3(sxyz, xyzT)
    m_count = xyzT.shape[1]
    iota = jax.lax.broadcasted_iota(jnp.int32, (sxyz.shape[0], m_count), 1)
    sel_w = jnp.zeros((sxyz.shape[0], m_count), jnp.float32)
    for r in range(3):
        sel_w = sel_w + jnp.where(iota == idx[:, r:r + 1], w[:, r:r + 1], 0.0)
    return _mm(sel_w, f2)


def _stages012_body(feats, xyz0T, sxyz0, sfeats0, xyz1T, sxyz1, sfeats1,
                    xyz2T, sxyz2, sfeats2,
                    u0g1, u0b1, u0w1, u0bb1, u0g2, u0b2, u0w2, u0bb2,
                    u1g1, u1b1, u1w1, u1bb1, u1g2, u1b2, u1w2, u1bb2,
                    u2g1, u2b1, u2w1, u2bb1, u2g2, u2b2, u2w2, u2bb2,
                    u3g2, u3b2, u3w2, u3bb2, f2_3_out):
    f = feats[...]
    params = (
        (xyz0T, sxyz0, sfeats0, u0g1, u0b1, u0w1, u0bb1, u0g2, u0b2, u0w2, u0bb2),
        (xyz1T, sxyz1, sfeats1, u1g1, u1b1, u1w1, u1bb1, u1g2, u1b2, u1w2, u1bb2),
        (xyz2T, sxyz2, sfeats2, u2g1, u2b1, u2w1, u2bb1, u2g2, u2b2, u2w2, u2bb2),
    )
    for (xyzT, sxyz, sfeats, sg1, sb1, sw1, sbb1, sg2, sb2, sw2, sbb2) in params:
        a = _mm(_ln(sfeats[...], sg1[...], sb1[...]), sw1[...]) + sbb1[...]
        f2 = _mm(_ln(f, sg2[...], sb2[...]), sw2[...]) + sbb2[...]
        f = a + _knn_mix(sxyz[...], xyzT[...], f2)
    f2_3_out[...] = _mm(_ln(f, u3g2[...], u3b2[...]), u3w2[...]) + u3bb2[...]


def _stage3_body(sxyz3, xyz3T, sfeats3, g1, b1, w1, bb1,
                 a_out, idx_out, w_out):
    a_out[...] = _mm(_ln(sfeats3[...], g1[...], b1[...]), w1[...]) + bb1[...]
    idx, w = _top3(sxyz3[...], xyz3T[...])
    idx_out[...] = idx
    w_out[...] = w


_P = 40
_NW = 32


def _sc_gather(idx_flat, f2):
    rows_total = idx_flat.shape[0]
    wr = rows_total // _NW
    sub = wr // (3 * _P)

    def body(idx_hbm, f2_hbm, out_hbm, idx_v, rows_v, gsem, ssem):
        wid = lax.axis_index("s") * 2 + lax.axis_index("c")
        base = wid * wr
        pltpu.sync_copy(idx_hbm.at[pl.ds(base, wr)], idx_v)
        gathers = []
        for s in range(sub):
            gathers.append(pltpu.async_copy(
                f2_hbm.at[idx_v.at[pl.ds(s * 3 * _P, 3 * _P)]],
                rows_v.at[pl.ds(s * 3 * _P, 3 * _P)], gsem))
        stores = []
        for s in range(sub):
            gathers[s].wait()
            stores.append(pltpu.async_copy(
                rows_v.at[pl.ds(s * 3 * _P, 3 * _P)],
                out_hbm.at[pl.ds(base + s * 3 * _P, 3 * _P)], ssem))
        for c in stores:
            c.wait()

    return functools.partial(
        pl.kernel,
        mesh=plsc.VectorSubcoreMesh(core_axis_name="c", subcore_axis_name="s"),
        out_type=jax.ShapeDtypeStruct((rows_total, 128), jnp.float32),
        scratch_types=[
            pltpu.VMEM((wr,), jnp.int32),
            pltpu.VMEM((wr, 128), jnp.float32),
            pltpu.SemaphoreType.DMA,
            pltpu.SemaphoreType.DMA,
        ],
    )(body)(idx_flat, f2)


def _combine_body(a_ref, rows_ref, w_ref, cw1, cb1, h_out):
    rows = rows_ref[...]
    w = w_ref[...]
    g = (w[:, 0:1] * rows[:, 0:128]
         + w[:, 1:2] * rows[:, 128:256]
         + w[:, 2:3] * rows[:, 256:384])
    f = a_ref[...] + g
    h_out[...] = _mm(f, cw1[...]) + cb1[...]


def _conf_body(h_ref, bn_g, bn_b, w2T, b2, conf_out):
    h = h_ref[...]
    mu = jnp.mean(h, axis=0, keepdims=True)
    var = jnp.mean((h - mu) ** 2, axis=0, keepdims=True)
    h = (h - mu) / jnp.sqrt(var + 1e-5) * bn_g[...] + bn_b[...]
    h = jnp.maximum(h, 0.0)
    conf_out[...] = jnp.sum(h * w2T[...], axis=1, keepdims=True) + b2[...]


_BN3 = 1000


def kernel(feats, xyz0, sxyz0, sfeats0, xyz1, sxyz1, sfeats1, xyz2, sxyz2, sfeats2, xyz3, sxyz3, sfeats3, u0_ln1_g, u0_ln1_b, u0_w1, u0_b1, u0_ln2_g, u0_ln2_b, u0_w2, u0_b2, u1_ln1_g, u1_ln1_b, u1_w1, u1_b1, u1_ln2_g, u1_ln2_b, u1_w2, u1_b2, u2_ln1_g, u2_ln1_b, u2_w1, u2_b1, u2_ln2_g, u2_ln2_b, u2_w2, u2_b2, u3_ln1_g, u3_ln1_b, u3_w1, u3_b1, u3_ln2_g, u3_ln2_b, u3_w2, u3_b2, conf_w1, conf_b1, conf_bn_g, conf_bn_b, conf_w2, conf_b2):
    r1 = lambda v: v.reshape(1, -1)
    n3 = sxyz3.shape[0]
    grid = (n3 // _BN3,)
    full = lambda s: pl.BlockSpec(s, lambda i: (0,) * len(s))
    blk = lambda s: pl.BlockSpec(s, lambda i: (i,) + (0,) * (len(s) - 1))

    f2_3 = pl.pallas_call(
        _stages012_body,
        out_shape=jax.ShapeDtypeStruct((2500, 128), jnp.float32),
    )(feats, xyz0.T, sxyz0, sfeats0, xyz1.T, sxyz1, sfeats1, xyz2.T, sxyz2,
      sfeats2,
      r1(u0_ln1_g), r1(u0_ln1_b), u0_w1, r1(u0_b1), r1(u0_ln2_g), r1(u0_ln2_b), u0_w2, r1(u0_b2),
      r1(u1_ln1_g), r1(u1_ln1_b), u1_w1, r1(u1_b1), r1(u1_ln2_g), r1(u1_ln2_b), u1_w2, r1(u1_b2),
      r1(u2_ln1_g), r1(u2_ln1_b), u2_w1, r1(u2_b1), r1(u2_ln2_g), r1(u2_ln2_b), u2_w2, r1(u2_b2),
      r1(u3_ln2_g), r1(u3_ln2_b), u3_w2, r1(u3_b2))

    a3, idx3, w3 = pl.pallas_call(
        _stage3_body,
        grid=grid,
        in_specs=[
            blk((_BN3, 3)),
            full((3, 2500)),
            blk((_BN3, 128)),
            full((1, 128)), full((1, 128)),
            full((128, 128)), full((1, 128)),
        ],
        out_specs=[blk((_BN3, 128)), blk((_BN3, 3)), blk((_BN3, 3))],
        out_shape=[jax.ShapeDtypeStruct((n3, 128), jnp.float32),
                   jax.ShapeDtypeStruct((n3, 3), jnp.int32),
                   jax.ShapeDtypeStruct((n3, 3), jnp.float32)],
    )(sxyz3, xyz3.T, sfeats3, r1(u3_ln1_g), r1(u3_ln1_b), u3_w1, r1(u3_b1))

    rows_pad = -(3 * n3) % (_NW * 3 * _P)
    npad = (3 * n3 + rows_pad) // 3
    idx_flat = jnp.pad(idx3.reshape(-1), (0, rows_pad))
    rows = _sc_gather(idx_flat, f2_3).reshape(npad, 384)

    h = pl.pallas_call(
        _combine_body,
        grid=grid,
        in_specs=[
            blk((_BN3, 128)),
            blk((_BN3, 384)),
            blk((_BN3, 3)),
            full((128, 128)), full((1, 128)),
        ],
        out_specs=blk((_BN3, 128)),
        out_shape=jax.ShapeDtypeStruct((n3, 128), jnp.float32),
    )(a3, rows, w3, conf_w1, r1(conf_b1))

    conf = pl.pallas_call(
        _conf_body,
        out_shape=jax.ShapeDtypeStruct((n3, 1), jnp.float32),
    )(h, r1(conf_bn_g), r1(conf_bn_b), conf_w2.T, r1(conf_b2))
    return conf


# speedup vs baseline: 1.0436x; 1.0049x over previous
import functools

import jax
import jax.numpy as jnp
from jax import lax
from jax.experimental import pallas as pl
from jax.experimental.pallas import tpu as pltpu
from jax.experimental.pallas import tpu_sc as plsc


def _ln(x, g, b):
    mu = jnp.mean(x, axis=-1, keepdims=True)
    var = jnp.mean((x - mu) ** 2, axis=-1, keepdims=True)
    return (x - mu) / jnp.sqrt(var + 1e-5) * g + b


def _mm(a, b):
    return jax.lax.dot_general(a, b, (((1,), (0,)), ((), ())),
                               preferred_element_type=jnp.float32)


def _top3(sxyz, xyzT):
    d = None
    for c in range(3):
        diff = sxyz[:, c:c + 1] - xyzT[c:c + 1, :]
        sq = diff * diff
        d = sq if d is None else d + sq
    m_count = float(xyzT.shape[1])
    iota = jax.lax.broadcasted_iota(jnp.int32, d.shape, 1).astype(jnp.float32)
    remaining = d
    idxs, ws = [], []
    wsum = None
    for r in range(3):
        mval = jnp.min(remaining, axis=1, keepdims=True)
        eq = remaining == mval
        idx = jnp.min(jnp.where(eq, iota, m_count), axis=1, keepdims=True)
        dist = jnp.sqrt(jnp.maximum(mval, 1e-10))
        w = 1.0 / (dist + 1e-8)
        wsum = w if r == 0 else wsum + w
        idxs.append(idx)
        ws.append(w)
        if r < 2:
            remaining = jnp.where(iota == idx, jnp.inf, remaining)
    return (jnp.concatenate(idxs, axis=1).astype(jnp.int32),
            jnp.concatenate([w / wsum for w in ws], axis=1))


def _knn_mix(sxyz, xyzT, f2):
    idx, w = _top3(sxyz, xyzT)
    m_count = xyzT.shape[1]
    iota = jax.lax.broadcasted_iota(jnp.int32, (sxyz.shape[0], m_count), 1)
    sel_w = jnp.zeros((sxyz.shape[0], m_count), jnp.float32)
    for r in range(3):
        sel_w = sel_w + jnp.where(iota == idx[:, r:r + 1], w[:, r:r + 1], 0.0)
    return _mm(sel_w, f2)


def _stages012_body(feats, xyz0T, sxyz0, sfeats0, xyz1T, sxyz1, sfeats1,
                    xyz2T, sxyz2, sfeats2,
                    u0g1, u0b1, u0w1, u0bb1, u0g2, u0b2, u0w2, u0bb2,
                    u1g1, u1b1, u1w1, u1bb1, u1g2, u1b2, u1w2, u1bb2,
                    u2g1, u2b1, u2w1, u2bb1, u2g2, u2b2, u2w2, u2bb2,
                    u3g2, u3b2, u3w2, u3bb2, f2_3_out):
    f = feats[...]
    params = (
        (xyz0T, sxyz0, sfeats0, u0g1, u0b1, u0w1, u0bb1, u0g2, u0b2, u0w2, u0bb2),
        (xyz1T, sxyz1, sfeats1, u1g1, u1b1, u1w1, u1bb1, u1g2, u1b2, u1w2, u1bb2),
        (xyz2T, sxyz2, sfeats2, u2g1, u2b1, u2w1, u2bb1, u2g2, u2b2, u2w2, u2bb2),
    )
    for (xyzT, sxyz, sfeats, sg1, sb1, sw1, sbb1, sg2, sb2, sw2, sbb2) in params:
        a = _mm(_ln(sfeats[...], sg1[...], sb1[...]), sw1[...]) + sbb1[...]
        f2 = _mm(_ln(f, sg2[...], sb2[...]), sw2[...]) + sbb2[...]
        f = a + _knn_mix(sxyz[...], xyzT[...], f2)
    f2_3_out[...] = _mm(_ln(f, u3g2[...], u3b2[...]), u3w2[...]) + u3bb2[...]


def _stage3_body(sxyz3, xyz3T, sfeats3, g1, b1, w1, bb1,
                 a_out, idx_out, w_out):
    a_out[...] = _mm(_ln(sfeats3[...], g1[...], b1[...]), w1[...]) + bb1[...]
    idx, w = _top3(sxyz3[...], xyz3T[...])
    idx_out[...] = idx
    w_out[...] = w


_P = 40
_NW = 32


def _sc_gather(idx_flat, f2):
    rows_total = idx_flat.shape[0]
    wr = rows_total // _NW
    sub = wr // (3 * _P)

    def body(idx_hbm, f2_hbm, out_hbm, idx_v, rows_v, gsem, ssem):
        wid = lax.axis_index("s") * 2 + lax.axis_index("c")
        base = wid * wr
        pltpu.sync_copy(idx_hbm.at[pl.ds(base, wr)], idx_v)
        gathers = []
        for s in range(sub):
            gathers.append(pltpu.async_copy(
                f2_hbm.at[idx_v.at[pl.ds(s * 3 * _P, 3 * _P)]],
                rows_v.at[pl.ds(s * 3 * _P, 3 * _P)], gsem))
        stores = []
        for s in range(sub):
            gathers[s].wait()
            stores.append(pltpu.async_copy(
                rows_v.at[pl.ds(s * 3 * _P, 3 * _P)],
                out_hbm.at[pl.ds(base + s * 3 * _P, 3 * _P)], ssem))
        for c in stores:
            c.wait()

    return functools.partial(
        pl.kernel,
        mesh=plsc.VectorSubcoreMesh(core_axis_name="c", subcore_axis_name="s"),
        out_type=jax.ShapeDtypeStruct((rows_total, 128), jnp.float32),
        scratch_types=[
            pltpu.VMEM((wr,), jnp.int32),
            pltpu.VMEM((wr, 128), jnp.float32),
            pltpu.SemaphoreType.DMA,
            pltpu.SemaphoreType.DMA,
        ],
    )(body)(idx_flat, f2)


def _head_body(a_ref, rows_ref, w_ref, cw1, cb1, bn_g, bn_b, w2T, b2,
               conf_out):
    rows = rows_ref[...]
    w = w_ref[...]
    g = (w[:, 0:1] * rows[:, 0:128]
         + w[:, 1:2] * rows[:, 128:256]
         + w[:, 2:3] * rows[:, 256:384])
    f = a_ref[...] + g
    h = _mm(f, cw1[...]) + cb1[...]
    mu = jnp.mean(h, axis=0, keepdims=True)
    var = jnp.mean((h - mu) ** 2, axis=0, keepdims=True)
    h = (h - mu) / jnp.sqrt(var + 1e-5) * bn_g[...] + bn_b[...]
    h = jnp.maximum(h, 0.0)
    conf_out[...] = jnp.sum(h * w2T[...], axis=1, keepdims=True) + b2[...]


_BN3 = 1000


def kernel(feats, xyz0, sxyz0, sfeats0, xyz1, sxyz1, sfeats1, xyz2, sxyz2, sfeats2, xyz3, sxyz3, sfeats3, u0_ln1_g, u0_ln1_b, u0_w1, u0_b1, u0_ln2_g, u0_ln2_b, u0_w2, u0_b2, u1_ln1_g, u1_ln1_b, u1_w1, u1_b1, u1_ln2_g, u1_ln2_b, u1_w2, u1_b2, u2_ln1_g, u2_ln1_b, u2_w1, u2_b1, u2_ln2_g, u2_ln2_b, u2_w2, u2_b2, u3_ln1_g, u3_ln1_b, u3_w1, u3_b1, u3_ln2_g, u3_ln2_b, u3_w2, u3_b2, conf_w1, conf_b1, conf_bn_g, conf_bn_b, conf_w2, conf_b2):
    r1 = lambda v: v.reshape(1, -1)
    n3 = sxyz3.shape[0]
    grid = (n3 // _BN3,)
    full = lambda s: pl.BlockSpec(s, lambda i: (0,) * len(s))
    blk = lambda s: pl.BlockSpec(s, lambda i: (i,) + (0,) * (len(s) - 1))

    f2_3 = pl.pallas_call(
        _stages012_body,
        out_shape=jax.ShapeDtypeStruct((2500, 128), jnp.float32),
    )(feats, xyz0.T, sxyz0, sfeats0, xyz1.T, sxyz1, sfeats1, xyz2.T, sxyz2,
      sfeats2,
      r1(u0_ln1_g), r1(u0_ln1_b), u0_w1, r1(u0_b1), r1(u0_ln2_g), r1(u0_ln2_b), u0_w2, r1(u0_b2),
      r1(u1_ln1_g), r1(u1_ln1_b), u1_w1, r1(u1_b1), r1(u1_ln2_g), r1(u1_ln2_b), u1_w2, r1(u1_b2),
      r1(u2_ln1_g), r1(u2_ln1_b), u2_w1, r1(u2_b1), r1(u2_ln2_g), r1(u2_ln2_b), u2_w2, r1(u2_b2),
      r1(u3_ln2_g), r1(u3_ln2_b), u3_w2, r1(u3_b2))

    a3, idx3, w3 = pl.pallas_call(
        _stage3_body,
        grid=grid,
        in_specs=[
            blk((_BN3, 3)),
            full((3, 2500)),
            blk((_BN3, 128)),
            full((1, 128)), full((1, 128)),
            full((128, 128)), full((1, 128)),
        ],
        out_specs=[blk((_BN3, 128)), blk((_BN3, 3)), blk((_BN3, 3))],
        out_shape=[jax.ShapeDtypeStruct((n3, 128), jnp.float32),
                   jax.ShapeDtypeStruct((n3, 3), jnp.int32),
                   jax.ShapeDtypeStruct((n3, 3), jnp.float32)],
    )(sxyz3, xyz3.T, sfeats3, r1(u3_ln1_g), r1(u3_ln1_b), u3_w1, r1(u3_b1))

    rows_pad = -(3 * n3) % (_NW * 3 * _P)
    npad = (3 * n3 + rows_pad) // 3
    idx_flat = jnp.pad(idx3.reshape(-1), (0, rows_pad))
    rows = _sc_gather(idx_flat, f2_3).reshape(npad, 384)

    conf = pl.pallas_call(
        _head_body,
        grid=(1,),
        in_specs=[
            blk((n3, 128)),
            blk((n3, 384)),
            blk((n3, 3)),
            full((128, 128)), full((1, 128)),
            full((1, 128)), full((1, 128)),
            full((1, 128)), full((1, 1)),
        ],
        out_specs=blk((n3, 1)),
        out_shape=jax.ShapeDtypeStruct((n3, 1), jnp.float32),
    )(a3, rows, w3, conf_w1, r1(conf_b1), r1(conf_bn_g), r1(conf_bn_b),
      conf_w2.T, r1(conf_b2))
    return conf
